# baseline (speedup 1.0000x reference)
import functools

import jax
import jax.numpy as jnp
from jax import lax
from jax.experimental import pallas as pl
from jax.experimental.pallas import tpu as pltpu
from jax.experimental.pallas import tpu_sc as plsc

D = 128
DE = 144
K = 64
NB = 4
NC = 2
NS = 16
NW = NC * NS
BLK = 256


def _make_edge_kernel(n_pad, ee_pad):
    cpt = ee_pad // (NW * K)
    assert cpt % NB == 0
    rows_per_tile = n_pad // NS
    mesh = plsc.VectorSubcoreMesh(core_axis_name="c", subcore_axis_name="s",
                                  num_cores=NC, num_subcores=NS)

    @functools.partial(
        pl.kernel,
        out_type=jax.ShapeDtypeStruct((NC, n_pad, DE), jnp.float32),
        mesh=mesh,
        compiler_params=pltpu.CompilerParams(use_tc_tiling_on_sc=False, needs_layout_passes=False),
        scratch_types=dict(
            sidx_v=pltpu.VMEM((NB, 2 * K), jnp.int32),
            dsc_v=pltpu.VMEM((NB, K), jnp.int32),
            adst_v=pltpu.VMEM((NB, K), jnp.float32),
            rows_v=pltpu.VMEM((NB, K, DE), jnp.float32),
            sem_i=[pltpu.SemaphoreType.DMA] * NB,
            sem_a=[pltpu.SemaphoreType.DMA] * NB,
            sem_r=[pltpu.SemaphoreType.DMA] * NB,
            sem_w=[pltpu.SemaphoreType.DMA] * NB,
            acc_sh=pltpu.VMEM_SHARED((n_pad, DE), jnp.float32),
        ),
    )
    def edge_kernel(sidx_hbm, adst_hbm, h_hbm, acc_out, *,
                    sidx_v, dsc_v, adst_v, rows_v, acc_sh,
                    sem_i, sem_a, sem_r, sem_w):
        cid = lax.axis_index("c")
        sid = lax.axis_index("s")
        wid = cid * NS + sid

        zero16 = jnp.zeros((16,), jnp.float32)

        def zbody(i, _):
            for v in range(DE // 16):
                rows_v[0, i, pl.ds(v * 16, 16)] = zero16
            return 0

        lax.fori_loop(0, K, zbody, 0)
        base_rows = sid * rows_per_tile
        for t in range(rows_per_tile // K):
            pltpu.sync_copy(rows_v.at[0],
                            acc_sh.at[pl.ds(base_rows + t * K, K)])
        plsc.subcore_barrier()

        def idx_fetch(t, b):
            base = (t * NW + wid) * (2 * K)
            pltpu.async_copy(sidx_hbm.at[pl.ds(base, 2 * K)], sidx_v.at[b],
                             sem_i[b])

        def wait_idx(b):
            pltpu.make_async_copy(sidx_hbm.at[pl.ds(0, 2 * K)],
                                  sidx_v.at[b], sem_i[b]).wait()

        def wait_scatter(b):
            pltpu.make_async_copy(rows_v.at[b], acc_sh.at[dsc_v.at[b]],
                                  sem_w[b]).wait()

        def gath(b):
            pltpu.async_copy(adst_hbm.at[sidx_v.at[b, pl.ds(K, K)]],
                             adst_v.at[b], sem_a[b])
            pltpu.async_copy(h_hbm.at[sidx_v.at[b, pl.ds(0, K)]],
                             rows_v.at[b], sem_r[b])

        iota16 = lax.iota(jnp.int32, 16)
        col128 = jnp.full((16,), D, jnp.int32)
        onehot0 = jnp.where(iota16 == 0, 1.0, 0.0).astype(jnp.float32)

        def consume(t, b):
            pltpu.make_async_copy(adst_hbm.at[sidx_v.at[b, pl.ds(K, K)]],
                                  adst_v.at[b], sem_a[b]).wait()
            for g in range(K // 16):
                dsc_v[b, pl.ds(g * 16, 16)] = sidx_v[b, pl.ds(K + g * 16, 16)]
            pltpu.make_async_copy(h_hbm.at[sidx_v.at[b, pl.ds(0, K)]],
                                  rows_v.at[b], sem_r[b]).wait()
            @pl.when(t + NB < cpt)
            def _():
                idx_fetch(t + NB, b)
            nb1 = (b + 1) % NB

            @pl.when(t + 1 < cpt)
            def _():
                @pl.when(t + 1 >= NB)
                def _():
                    wait_scatter(nb1)
                wait_idx(nb1)
                gath(nb1)

            @plsc.parallel_loop(0, K // 16, unroll=2)
            def _(g):
                jbase = g * 16
                asrc16 = plsc.load_gather(rows_v.at[b],
                                          [jbase + iota16, col128])
                a = asrc16 + adst_v[b, pl.ds(jbase, 16)]
                e = jnp.maximum(a, 0.0) + 0.2 * jnp.minimum(a, 0.0)
                ex16 = jnp.exp(e)
                for j in range(16):
                    sc = ex16[j]
                    for v in range(D // 16):
                        rows_v[b, jbase + j, pl.ds(v * 16, 16)] = (
                            rows_v[b, jbase + j, pl.ds(v * 16, 16)] * sc)
                    rows_v[b, jbase + j, pl.ds(D, 16)] = sc * onehot0
            pltpu.async_copy(rows_v.at[b], acc_sh.at[dsc_v.at[b]], sem_w[b],
                             add=True)

        for t0 in range(NB):
            idx_fetch(t0, t0)
        wait_idx(0)
        gath(0)

        def loop_body(i, _):
            for sl in range(NB):
                consume(NB * i + sl, sl)
            return 0

        lax.fori_loop(0, cpt // NB, loop_body, 0)
        for b in range(NB):
            wait_scatter(b)
        plsc.subcore_barrier()

        pltpu.sync_copy(acc_sh.at[pl.ds(base_rows, rows_per_tile)],
                        acc_out.at[cid, pl.ds(base_rows, rows_per_tile)])

    return edge_kernel


def _swish(z):
    return z / (1.0 + jnp.exp(-z))


def _mm_first_body(x_ref, w_ref, he_ref):
    he_ref[...] = jnp.dot(x_ref[...], w_ref[...],
                          preferred_element_type=jnp.float32)


def _mm_first(x_pad, wext, n_pad):
    return pl.pallas_call(
        _mm_first_body,
        grid=(n_pad // BLK,),
        in_specs=[
            pl.BlockSpec((BLK, D), lambda i: (i, 0)),
            pl.BlockSpec((D, DE), lambda i: (0, 0)),
        ],
        out_specs=pl.BlockSpec((BLK, DE), lambda i: (i, 0)),
        out_shape=jax.ShapeDtypeStruct((n_pad, DE), jnp.float32),
    )(x_pad, wext)


def _gat_epilogue(acc_ref, b_ref):
    accb = acc_ref[...]
    num = accb[0, :, :D] + accb[1, :, :D]
    ssum = accb[0, :, D] + accb[1, :, D]
    y = num / (ssum[:, None] + 1e-30) + b_ref[...]
    return _swish(y)


def _ep_mid_body(acc_ref, b_ref, w_ref, he_ref):
    y = _gat_epilogue(acc_ref, b_ref)
    he_ref[...] = jnp.dot(y, w_ref[...], preferred_element_type=jnp.float32)


def _ep_mid(acc, b, wext, n_pad):
    return pl.pallas_call(
        _ep_mid_body,
        grid=(n_pad // BLK,),
        in_specs=[
            pl.BlockSpec((NC, BLK, DE), lambda i: (0, i, 0)),
            pl.BlockSpec((1, D), lambda i: (0, 0)),
            pl.BlockSpec((D, DE), lambda i: (0, 0)),
        ],
        out_specs=pl.BlockSpec((BLK, DE), lambda i: (i, 0)),
        out_shape=jax.ShapeDtypeStruct((n_pad, DE), jnp.float32),
    )(acc, b, wext)


def _ep_final_body(acc_ref, b_ref, w_ref, bfc_ref, out_ref):
    y = _gat_epilogue(acc_ref, b_ref)
    out_ref[...] = (jnp.dot(y, w_ref[...], preferred_element_type=jnp.float32)
                    + bfc_ref[...])


def _ep_final(acc, b, wfc_t, bfc, n_pad):
    return pl.pallas_call(
        _ep_final_body,
        grid=(n_pad // BLK,),
        in_specs=[
            pl.BlockSpec((NC, BLK, DE), lambda i: (0, i, 0)),
            pl.BlockSpec((1, D), lambda i: (0, 0)),
            pl.BlockSpec((D, D), lambda i: (0, 0)),
            pl.BlockSpec((1, D), lambda i: (0, 0)),
        ],
        out_specs=pl.BlockSpec((BLK, D), lambda i: (i, 0)),
        out_shape=jax.ShapeDtypeStruct((n_pad, D), jnp.float32),
    )(acc, b, wfc_t, bfc)


def kernel(batch_x, batch_edge_index, W1, att_src1, att_dst1, b1,
           W2, att_src2, att_dst2, b2, Wfc, bfc):
    n = batch_x.shape[1]
    e = batch_edge_index.shape[2]
    ee = e + n
    n_pad = ((n + BLK - 1) // BLK) * BLK
    ee_pad = ((ee + NB * NW * K - 1) // (NB * NW * K)) * (NB * NW * K)

    x = batch_x[0]
    ei = batch_edge_index[0]
    loops = jnp.arange(n, dtype=jnp.int32)
    npad_e = ee_pad - ee
    pad_src = jnp.full((npad_e,), n, dtype=jnp.int32)
    pad_dst = n + (jnp.arange(npad_e, dtype=jnp.int32) % (n_pad - n))
    src = jnp.concatenate([ei[0].astype(jnp.int32), loops, pad_src])
    dst = jnp.concatenate([ei[1].astype(jnp.int32), loops, pad_dst])
    sidx = jnp.concatenate(
        [src.reshape(-1, K), dst.reshape(-1, K)], axis=1).reshape(-1)

    x_pad = jnp.zeros((n_pad, D), jnp.float32).at[:n].set(x)

    def _wext(W, a_s, a_d):
        return jnp.concatenate(
            [W, (W @ a_s)[:, None], (W @ a_d)[:, None],
             jnp.zeros((D, DE - D - 2), jnp.float32)], axis=1)

    edge_kernel = _make_edge_kernel(n_pad, ee_pad)

    h1e = _mm_first(x_pad, _wext(W1, att_src1, att_dst1), n_pad)
    acc1 = edge_kernel(sidx, h1e[:, D + 1], h1e)

    h2e = _ep_mid(acc1, b1.reshape(1, D), _wext(W2, att_src2, att_dst2),
                  n_pad)
    acc2 = edge_kernel(sidx, h2e[:, D + 1], h2e)

    out = _ep_final(acc2, b2.reshape(1, D), Wfc.T, bfc.reshape(1, D), n_pad)
    return out[:n][None, :, :]


# speedup vs baseline: 1.4259x; 1.4259x over previous
import functools

import jax
import jax.numpy as jnp
from jax import lax
from jax.experimental import pallas as pl
from jax.experimental.pallas import tpu as pltpu
from jax.experimental.pallas import tpu_sc as plsc

D = 128
DE = 144
K = 64
NB = 2
NC = 2
NS = 16
NW = NC * NS
BLK = 256


def _make_edge_kernel(n_pad, ee_pad):
    cpt = ee_pad // (NW * K)
    assert cpt % NB == 0
    rows_per_tile = n_pad // NS
    mesh = plsc.VectorSubcoreMesh(core_axis_name="c", subcore_axis_name="s",
                                  num_cores=NC, num_subcores=NS)

    @functools.partial(
        pl.kernel,
        out_type=jax.ShapeDtypeStruct((NC, n_pad, DE), jnp.float32),
        mesh=mesh,
        compiler_params=pltpu.CompilerParams(use_tc_tiling_on_sc=False, needs_layout_passes=False),
        scratch_types=dict(
            sidx_v=pltpu.VMEM((NB, 2 * K), jnp.int32),
            dsc_v=pltpu.VMEM((NB, K), jnp.int32),
            adst_v=pltpu.VMEM((NB, K), jnp.float32),
            rows_v=pltpu.VMEM((NB, K, DE), jnp.float32),
            sem_i=[pltpu.SemaphoreType.DMA] * NB,
            sem_a=[pltpu.SemaphoreType.DMA] * NB,
            sem_r=[pltpu.SemaphoreType.DMA] * NB,
            sem_w=[pltpu.SemaphoreType.DMA] * NB,
            acc_sh=pltpu.VMEM_SHARED((n_pad, DE), jnp.float32),
        ),
    )
    def edge_kernel(sidx_hbm, adst_hbm, h_hbm, acc_out, *,
                    sidx_v, dsc_v, adst_v, rows_v, acc_sh,
                    sem_i, sem_a, sem_r, sem_w):
        cid = lax.axis_index("c")
        sid = lax.axis_index("s")
        wid = cid * NS + sid

        zero16 = jnp.zeros((16,), jnp.float32)

        def zbody(i, _):
            for v in range(DE // 16):
                rows_v[0, i, pl.ds(v * 16, 16)] = zero16
            return 0

        lax.fori_loop(0, K, zbody, 0)
        base_rows = sid * rows_per_tile
        for t in range(rows_per_tile // K):
            pltpu.sync_copy(rows_v.at[0],
                            acc_sh.at[pl.ds(base_rows + t * K, K)])
        plsc.subcore_barrier()

        def idx_fetch(t, b):
            base = (t * NW + wid) * (2 * K)
            pltpu.async_copy(sidx_hbm.at[pl.ds(base, 2 * K)], sidx_v.at[b],
                             sem_i[b])

        def wait_idx(b):
            pltpu.make_async_copy(sidx_hbm.at[pl.ds(0, 2 * K)],
                                  sidx_v.at[b], sem_i[b]).wait()

        def wait_scatter(b):
            pltpu.make_async_copy(rows_v.at[b], acc_sh.at[dsc_v.at[b]],
                                  sem_w[b]).wait()

        def gath(b):
            pltpu.async_copy(adst_hbm.at[sidx_v.at[b, pl.ds(K, K)]],
                             adst_v.at[b], sem_a[b])
            pltpu.async_copy(h_hbm.at[sidx_v.at[b, pl.ds(0, K)]],
                             rows_v.at[b], sem_r[b])

        iota16 = lax.iota(jnp.int32, 16)
        col128 = jnp.full((16,), D, jnp.int32)
        onehot0 = jnp.where(iota16 == 0, 1.0, 0.0).astype(jnp.float32)

        def consume(t, b):
            pltpu.make_async_copy(adst_hbm.at[sidx_v.at[b, pl.ds(K, K)]],
                                  adst_v.at[b], sem_a[b]).wait()
            for g in range(K // 16):
                dsc_v[b, pl.ds(g * 16, 16)] = sidx_v[b, pl.ds(K + g * 16, 16)]
            pltpu.make_async_copy(h_hbm.at[sidx_v.at[b, pl.ds(0, K)]],
                                  rows_v.at[b], sem_r[b]).wait()
            @pl.when(t + NB < cpt)
            def _():
                idx_fetch(t + NB, b)
            nb1 = (b + 1) % NB

            @pl.when(t + 1 < cpt)
            def _():
                @pl.when(t + 1 >= NB)
                def _():
                    wait_scatter(nb1)
                wait_idx(nb1)
                gath(nb1)

            @plsc.parallel_loop(0, K // 16, unroll=2)
            def _(g):
                jbase = g * 16
                asrc16 = plsc.load_gather(rows_v.at[b],
                                          [jbase + iota16, col128])
                a = asrc16 + adst_v[b, pl.ds(jbase, 16)]
                e = jnp.maximum(a, 0.0) + 0.2 * jnp.minimum(a, 0.0)
                ex16 = jnp.exp(e)
                for j in range(16):
                    sc = ex16[j]
                    for v in range(D // 16):
                        rows_v[b, jbase + j, pl.ds(v * 16, 16)] = (
                            rows_v[b, jbase + j, pl.ds(v * 16, 16)] * sc)
                    rows_v[b, jbase + j, pl.ds(D, 16)] = sc * onehot0
            pltpu.async_copy(rows_v.at[b], acc_sh.at[dsc_v.at[b]], sem_w[b],
                             add=True)

        for t0 in range(NB):
            idx_fetch(t0, t0)
        wait_idx(0)
        gath(0)

        def loop_body(i, _):
            for sl in range(NB):
                consume(NB * i + sl, sl)
            return 0

        lax.fori_loop(0, cpt // NB, loop_body, 0)
        for b in range(NB):
            wait_scatter(b)
        plsc.subcore_barrier()

        pltpu.sync_copy(acc_sh.at[pl.ds(base_rows, rows_per_tile)],
                        acc_out.at[cid, pl.ds(base_rows, rows_per_tile)])

    return edge_kernel


def _swish(z):
    return z / (1.0 + jnp.exp(-z))


def _mm_first_body(x_ref, w_ref, he_ref):
    he_ref[...] = jnp.dot(x_ref[...], w_ref[...],
                          preferred_element_type=jnp.float32)


def _mm_first(x_pad, wext, n_pad):
    return pl.pallas_call(
        _mm_first_body,
        grid=(n_pad // BLK,),
        in_specs=[
            pl.BlockSpec((BLK, D), lambda i: (i, 0)),
            pl.BlockSpec((D, DE), lambda i: (0, 0)),
        ],
        out_specs=pl.BlockSpec((BLK, DE), lambda i: (i, 0)),
        out_shape=jax.ShapeDtypeStruct((n_pad, DE), jnp.float32),
    )(x_pad, wext)


def _gat_epilogue(acc_ref, b_ref):
    accb = acc_ref[...]
    num = accb[0, :, :D] + accb[1, :, :D]
    ssum = accb[0, :, D] + accb[1, :, D]
    y = num / (ssum[:, None] + 1e-30) + b_ref[...]
    return _swish(y)


def _ep_mid_body(acc_ref, b_ref, w_ref, he_ref):
    y = _gat_epilogue(acc_ref, b_ref)
    he_ref[...] = jnp.dot(y, w_ref[...], preferred_element_type=jnp.float32)


def _ep_mid(acc, b, wext, n_pad):
    return pl.pallas_call(
        _ep_mid_body,
        grid=(n_pad // BLK,),
        in_specs=[
            pl.BlockSpec((NC, BLK, DE), lambda i: (0, i, 0)),
            pl.BlockSpec((1, D), lambda i: (0, 0)),
            pl.BlockSpec((D, DE), lambda i: (0, 0)),
        ],
        out_specs=pl.BlockSpec((BLK, DE), lambda i: (i, 0)),
        out_shape=jax.ShapeDtypeStruct((n_pad, DE), jnp.float32),
    )(acc, b, wext)


def _ep_final_body(acc_ref, b_ref, w_ref, bfc_ref, out_ref):
    y = _gat_epilogue(acc_ref, b_ref)
    out_ref[...] = (jnp.dot(y, w_ref[...], preferred_element_type=jnp.float32)
                    + bfc_ref[...])


def _ep_final(acc, b, wfc_t, bfc, n_pad):
    return pl.pallas_call(
        _ep_final_body,
        grid=(n_pad // BLK,),
        in_specs=[
            pl.BlockSpec((NC, BLK, DE), lambda i: (0, i, 0)),
            pl.BlockSpec((1, D), lambda i: (0, 0)),
            pl.BlockSpec((D, D), lambda i: (0, 0)),
            pl.BlockSpec((1, D), lambda i: (0, 0)),
        ],
        out_specs=pl.BlockSpec((BLK, D), lambda i: (i, 0)),
        out_shape=jax.ShapeDtypeStruct((n_pad, D), jnp.float32),
    )(acc, b, wfc_t, bfc)


def kernel(batch_x, batch_edge_index, W1, att_src1, att_dst1, b1,
           W2, att_src2, att_dst2, b2, Wfc, bfc):
    n = batch_x.shape[1]
    e = batch_edge_index.shape[2]
    ee = e + n
    n_pad = ((n + BLK - 1) // BLK) * BLK
    ee_pad = ((ee + NB * NW * K - 1) // (NB * NW * K)) * (NB * NW * K)

    x = batch_x[0]
    ei = batch_edge_index[0]
    loops = jnp.arange(n, dtype=jnp.int32)
    npad_e = ee_pad - ee
    pad_src = jnp.full((npad_e,), n, dtype=jnp.int32)
    pad_dst = n + (jnp.arange(npad_e, dtype=jnp.int32) % (n_pad - n))
    src = jnp.concatenate([ei[0].astype(jnp.int32), loops, pad_src])
    dst = jnp.concatenate([ei[1].astype(jnp.int32), loops, pad_dst])
    sidx = jnp.concatenate(
        [src.reshape(-1, K), dst.reshape(-1, K)], axis=1).reshape(-1)

    x_pad = jnp.zeros((n_pad, D), jnp.float32).at[:n].set(x)

    def _wext(W, a_s, a_d):
        return jnp.concatenate(
            [W, (W @ a_s)[:, None], (W @ a_d)[:, None],
             jnp.zeros((D, DE - D - 2), jnp.float32)], axis=1)

    edge_kernel = _make_edge_kernel(n_pad, ee_pad)

    h1e = _mm_first(x_pad, _wext(W1, att_src1, att_dst1), n_pad)
    acc1 = edge_kernel(sidx, h1e[:, D + 1], h1e)

    h2e = _ep_mid(acc1, b1.reshape(1, D), _wext(W2, att_src2, att_dst2),
                  n_pad)
    acc2 = edge_kernel(sidx, h2e[:, D + 1], h2e)

    out = _ep_final(acc2, b2.reshape(1, D), Wfc.T, bfc.reshape(1, D), n_pad)
    return out[:n][None, :, :]
